# baseline (speedup 1.0000x reference)
import jax
import jax.numpy as jnp
from jax import lax
from jax.experimental import pallas as pl
from jax.experimental.pallas import tpu as pltpu
from jax.experimental.pallas import tpu_sc as plsc

B, L, H = 4, 8192, 1024
NC, NS = 2, 16
NW = NC * NS
LPW = L // NW
C = 16
NCH = LPW // C
HV = H // 16


def _sc_body(x_hbm, emb_hbm, o_hbm, ebuf, xb0, xb1, xb2, xb3):
    w = lax.axis_index("s") * NC + lax.axis_index("c")
    l_tile = w * LPW
    xbufs = (xb0, xb1, xb2, xb3)

    def chunk(j, carry):
        l0 = l_tile + j * C
        pltpu.sync_copy(emb_hbm.at[pl.ds(l0, C)], ebuf)
        for b in range(B):
            pltpu.sync_copy(x_hbm.at[b, pl.ds(l0, C)], xbufs[b])

        def row(r, rc):
            for k in range(HV):
                e = ebuf[r, pl.ds(k * 16, 16)]
                for b in range(B):
                    plsc.addupdate(xbufs[b].at[r, pl.ds(k * 16, 16)], e)
            return rc

        lax.fori_loop(0, C, row, 0)
        for b in range(B):
            pltpu.sync_copy(xbufs[b], o_hbm.at[b, pl.ds(l0, C)])
        return carry

    lax.fori_loop(0, NCH, chunk, 0)


_run = pl.kernel(
    _sc_body,
    out_type=jax.ShapeDtypeStruct((B, L, H), jnp.float32),
    mesh=plsc.VectorSubcoreMesh(core_axis_name="c", subcore_axis_name="s"),
    scratch_types=[pltpu.VMEM((C, H), jnp.float32)] * 5,
)


def kernel(x, pos_emb):
    return _run(x, pos_emb)


# speedup vs baseline: 1.9051x; 1.9051x over previous
import jax
import jax.numpy as jnp
from jax import lax
from jax.experimental import pallas as pl
from jax.experimental.pallas import tpu as pltpu
from jax.experimental.pallas import tpu_sc as plsc

B, L, H = 4, 8192, 1024
NC, NS = 2, 16
NW = NC * NS
LPW = L // NW
C = 8
NCH = LPW // C
NG = NCH // 2
HV = H // 16


def _sc_body(x_hbm, emb_hbm, o_hbm, ebuf, xb, si0, si1, so0, so1):
    w = lax.axis_index("s") * NC + lax.axis_index("c")
    l_tile = w * LPW
    si = (si0, si1)
    so = (so0, so1)

    def in_descs(j, p):
        l0 = l_tile + j * C
        d = [pltpu.make_async_copy(emb_hbm.at[pl.ds(l0, C)], ebuf.at[p], si[p])]
        for b in range(B):
            d.append(
                pltpu.make_async_copy(x_hbm.at[b, pl.ds(l0, C)], xb.at[p, b], si[p])
            )
        return d

    def out_descs(j, p):
        l0 = l_tile + j * C
        return [
            pltpu.make_async_copy(xb.at[p, b], o_hbm.at[b, pl.ds(l0, C)], so[p])
            for b in range(B)
        ]

    def compute(p):
        def row(r, rc):
            for k in range(HV):
                e = ebuf[p, r, pl.ds(k * 16, 16)]
                for b in range(B):
                    plsc.addupdate(xb.at[p, b, r, pl.ds(k * 16, 16)], e)
            return rc

        lax.fori_loop(0, C, row, 0)

    for d in in_descs(0, 0):
        d.start()

    def g_body(g, carry):
        for p in (0, 1):
            j = 2 * g + p
            for d in in_descs(j, p):
                d.wait()
            jn = j + 1

            @pl.when(jn < NCH)
            def _start_next():
                @pl.when(jn >= 2)
                def _drain_prev_out():
                    for d in out_descs(jn - 2, 1 - p):
                        d.wait()

                for d in in_descs(jn, 1 - p):
                    d.start()

            compute(p)
            for d in out_descs(j, p):
                d.start()
        return carry

    lax.fori_loop(0, NG, g_body, 0)
    for d in out_descs(NCH - 2, 0):
        d.wait()
    for d in out_descs(NCH - 1, 1):
        d.wait()


_run = pl.kernel(
    _sc_body,
    out_type=jax.ShapeDtypeStruct((B, L, H), jnp.float32),
    mesh=plsc.VectorSubcoreMesh(core_axis_name="c", subcore_axis_name="s"),
    scratch_types=[
        pltpu.VMEM((2, C, H), jnp.float32),
        pltpu.VMEM((2, B, C, H), jnp.float32),
        pltpu.SemaphoreType.DMA,
        pltpu.SemaphoreType.DMA,
        pltpu.SemaphoreType.DMA,
        pltpu.SemaphoreType.DMA,
    ],
)


def kernel(x, pos_emb):
    return _run(x, pos_emb)
